# baseline (speedup 1.0000x reference)
import functools

import jax
import jax.numpy as jnp
from jax import lax
from jax.experimental import pallas as pl
from jax.experimental.pallas import tpu as pltpu, tpu_sc as plsc

NC = 2
NS = 16
NW = NC * NS

D = 128
DH = D // 2
CHUNK = 128
CA = 256
CW = 8


def _sc_agg(n_rows, n_chunks):
    rows_per_tile = n_rows // NS
    mesh = plsc.VectorSubcoreMesh(core_axis_name="c", subcore_axis_name="s")
    S = 4
    LG = 2
    LI = 3
    assert n_chunks % S == 0

    @functools.partial(
        pl.kernel,
        out_type=jax.ShapeDtypeStruct((NC * n_rows, DH), jnp.float32),
        mesh=mesh,
        scratch_types=[
            [pltpu.VMEM((2, CA), jnp.int32)] * S,
            [pltpu.VMEM((CA, DH), jnp.float32)] * S,
            pltpu.VMEM_SHARED((n_rows, DH), jnp.float32),
            [pltpu.SemaphoreType.DMA] * S,
            [pltpu.SemaphoreType.DMA] * S,
            [pltpu.SemaphoreType.DMA] * S,
        ],
        compiler_params=pltpu.CompilerParams(use_tc_tiling_on_sc=False))
    def body(h_hbm, idx_hbm, zeros_hbm, agg_out,
             idxs, rows, acc_sh, isem, gsem, ssem):
        cid = lax.axis_index("c")
        sid = lax.axis_index("s")
        idx_base = sid * n_chunks
        row_base = sid * rows_per_tile

        pltpu.sync_copy(zeros_hbm.at[pl.ds(0, rows_per_tile)],
                        acc_sh.at[pl.ds(row_base, rows_per_tile)])

        def fetch_idx(c, k):
            pltpu.async_copy(idx_hbm.at[cid, idx_base + c], idxs[k], isem[k])

        def wait_idx(k):
            pltpu.make_async_copy(idx_hbm.at[0, 0], idxs[k], isem[k]).wait()

        def gather(c, k):
            pltpu.async_copy(h_hbm.at[idxs[k].at[0]], rows[k], gsem[k])

        def scatter(k):
            pltpu.async_copy(rows[k], acc_sh.at[idxs[k].at[1]], ssem[k],
                             add=True)

        for c in range(LI):
            fetch_idx(c, c)
        for c in range(LG):
            wait_idx(c)
            gather(c, c)
        plsc.subcore_barrier()

        @pl.loop(0, n_chunks, step=S)
        def _(g):
            for k in range(S):
                c = g + k
                ki = (k + LI) % S

                @pl.when(c + LI < n_chunks)
                def _():
                    @pl.when(c + LI - S >= 0)
                    def _():
                        pltpu.make_async_copy(rows[ki],
                                              acc_sh.at[idxs[ki].at[1]],
                                              ssem[ki]).wait()
                    fetch_idx(c + LI, ki)

                kg = (k + LG) % S

                @pl.when(c + LG < n_chunks)
                def _():
                    wait_idx(kg)
                    gather(c + LG, kg)

                pltpu.make_async_copy(h_hbm.at[idxs[k].at[0]], rows[k],
                                      gsem[k]).wait()
                scatter(k)

        for k in range(S):
            pltpu.make_async_copy(rows[k], acc_sh.at[idxs[k].at[1]],
                                  ssem[k]).wait()

        plsc.subcore_barrier()

        pltpu.sync_copy(
            acc_sh.at[pl.ds(row_base, rows_per_tile)],
            agg_out.at[pl.ds(cid * n_rows + row_base, rows_per_tile)])

    return body


def _sc_counts(n_rows, n_chunks):
    rows_per_tile = n_rows // NS
    mesh = plsc.VectorSubcoreMesh(core_axis_name="c", subcore_axis_name="s")

    @functools.partial(
        pl.kernel,
        out_type=jax.ShapeDtypeStruct((NC, n_rows, CW), jnp.float32),
        mesh=mesh,
        scratch_types=[
            pltpu.VMEM((n_chunks, CHUNK), jnp.int32),
            pltpu.VMEM((CHUNK, CW), jnp.float32),
            pltpu.VMEM_SHARED((n_rows, CW), jnp.float32),
        ],
        compiler_params=pltpu.CompilerParams(use_tc_tiling_on_sc=False))
    def body(dst_hbm, zc_hbm, ones_hbm, cnt_out, dst_v, ones_v, cnt_sh):
        cid = lax.axis_index("c")
        sid = lax.axis_index("s")
        wid = cid * NS + sid
        idx_base = wid * n_chunks
        row_base = sid * rows_per_tile

        pltpu.sync_copy(dst_hbm.at[pl.ds(idx_base, n_chunks)], dst_v)
        pltpu.sync_copy(ones_hbm, ones_v)
        pltpu.sync_copy(zc_hbm, cnt_sh.at[pl.ds(row_base, rows_per_tile)])
        plsc.subcore_barrier()

        @pl.loop(0, n_chunks)
        def _(j):
            pltpu.sync_copy(ones_v, cnt_sh.at[dst_v.at[j]], add=True)

        plsc.subcore_barrier()
        pltpu.sync_copy(cnt_sh.at[pl.ds(row_base, rows_per_tile)],
                        cnt_out.at[cid, pl.ds(row_base, rows_per_tile)])

    return body


def _tc_layer(n_rows, block):
    nb = n_rows // block

    def body(alo, ahi, xlo, xhi, cnt_ref, wl_ref, wr_ref, b_ref, o_ref):
        agg = jnp.concatenate([alo[...], ahi[...]], axis=1)
        x = jnp.concatenate([xlo[...], xhi[...]], axis=1)
        cnt = cnt_ref[0, :, 0:1] + cnt_ref[1, :, 0:1]
        mean = agg / jnp.maximum(cnt, 1.0)
        o_ref[...] = jnp.maximum(
            jnp.dot(mean, wl_ref[...], preferred_element_type=jnp.float32)
            + jnp.dot(x, wr_ref[...], preferred_element_type=jnp.float32)
            + b_ref[0:1, :], 0.0)

    half = pl.BlockSpec((block, DH), lambda i, j: (i, 0))
    other = pl.BlockSpec((block, DH), lambda i, j: (nb + i, 0))
    return pl.pallas_call(
        body,
        grid=(nb, 2),
        in_specs=[
            half, other,
            half, other,
            pl.BlockSpec((NC, block, CW), lambda i, j: (0, i, 0)),
            pl.BlockSpec((D, DH), lambda i, j: (j, 0)),
            pl.BlockSpec((D, DH), lambda i, j: (j, 0)),
            pl.BlockSpec((8, DH), lambda i, j: (j, 0)),
        ],
        out_specs=pl.BlockSpec((block, DH), lambda i, j: (j * nb + i, 0)),
        out_shape=jax.ShapeDtypeStruct((NC * n_rows, DH), jnp.float32),
    )


def kernel(x, edge_index, W_l0, b_l0, W_r0, W_l1, b_l1, W_r1):
    n, d = x.shape
    e = edge_index.shape[1]

    nc_cnt = -(-e // (NW * CHUNK))
    nc_cnt = -(-nc_cnt // 8) * 8
    nc_agg = nc_cnt
    e_pad = NW * nc_cnt * CHUNK
    assert e_pad == NS * nc_agg * CA
    rows_per_tile = -(-n // NW)
    rows_per_tile = -(-rows_per_tile // CHUNK) * CHUNK
    n_rows = rows_per_tile * NW

    src = edge_index[0].astype(jnp.int32)
    dst = edge_index[1].astype(jnp.int32)
    src = jnp.pad(src, (0, e_pad - e))
    dst = jnp.pad(dst, (0, e_pad - e), constant_values=n_rows - 1)
    srcc = src.reshape(NS * nc_agg, 1, CA)
    dstc = dst.reshape(NS * nc_agg, 1, CA)
    idxp = jnp.stack([
        jnp.concatenate([srcc, dstc], axis=1),
        jnp.concatenate([srcc + n_rows, dstc], axis=1),
    ])
    dst2d = dst.reshape(NW * nc_cnt, CHUNK)

    zeros = jnp.zeros((n_rows // NS, DH), jnp.float32)
    zeros_cnt = jnp.zeros((n_rows // NS, CW), jnp.float32)
    ones = jnp.ones((CHUNK, CW), jnp.float32)

    x_pad = jnp.pad(x, ((0, n_rows - n), (0, 0)))
    x_cat = jnp.concatenate([x_pad[:, :DH], x_pad[:, DH:]], axis=0)

    def colsplit(w):
        return jnp.concatenate([w[:, :DH], w[:, DH:]], axis=0)

    wl0, wr0 = colsplit(W_l0), colsplit(W_r0)
    wl1, wr1 = colsplit(W_l1), colsplit(W_r1)
    def biassplit(b):
        return jnp.pad(b.reshape(2, 1, DH),
                       ((0, 0), (0, 7), (0, 0))).reshape(16, DH)

    bc0 = biassplit(b_l0)
    bc1 = biassplit(b_l1)

    sc_agg = _sc_agg(n_rows, nc_agg)
    sc_counts = _sc_counts(n_rows, nc_cnt)
    tc = _tc_layer(n_rows, 512)

    cnt = sc_counts(dst2d, zeros_cnt, ones)
    agg0 = sc_agg(x_cat, idxp, zeros)
    h1 = tc(agg0, agg0, x_cat, x_cat, cnt, wl0, wr0, bc0)
    agg1 = sc_agg(h1, idxp, zeros)
    h2 = tc(agg1, agg1, h1, h1, cnt, wl1, wr1, bc1)
    return jnp.concatenate([h2[:n], h2[n_rows:n_rows + n]], axis=1)


# speedup vs baseline: 1.0755x; 1.0755x over previous
import functools

import jax
import jax.numpy as jnp
from jax import lax
from jax.experimental import pallas as pl
from jax.experimental.pallas import tpu as pltpu, tpu_sc as plsc

NC = 2
NS = 16
NW = NC * NS

D = 128
DH = D // 2
CHUNK = 128
CA = 128
CW = 8


def _sc_agg(n_rows, n_chunks):
    rows_per_tile = n_rows // NS
    mesh = plsc.VectorSubcoreMesh(core_axis_name="c", subcore_axis_name="s")
    S = 8
    LG = 3
    LI = 6
    assert n_chunks % S == 0

    @functools.partial(
        pl.kernel,
        out_type=jax.ShapeDtypeStruct((NC * n_rows, DH), jnp.float32),
        mesh=mesh,
        scratch_types=[
            [pltpu.VMEM((2, CA), jnp.int32)] * S,
            [pltpu.VMEM((CA, DH), jnp.float32)] * S,
            pltpu.VMEM_SHARED((n_rows, DH), jnp.float32),
            [pltpu.SemaphoreType.DMA] * S,
            [pltpu.SemaphoreType.DMA] * S,
            [pltpu.SemaphoreType.DMA] * S,
        ],
        compiler_params=pltpu.CompilerParams(use_tc_tiling_on_sc=False))
    def body(h_hbm, idx_hbm, zeros_hbm, agg_out,
             idxs, rows, acc_sh, isem, gsem, ssem):
        cid = lax.axis_index("c")
        sid = lax.axis_index("s")
        idx_base = sid * n_chunks
        row_base = sid * rows_per_tile

        pltpu.sync_copy(zeros_hbm.at[pl.ds(0, rows_per_tile)],
                        acc_sh.at[pl.ds(row_base, rows_per_tile)])

        def fetch_idx(c, k):
            pltpu.async_copy(idx_hbm.at[cid, idx_base + c], idxs[k], isem[k])

        def wait_idx(k):
            pltpu.make_async_copy(idx_hbm.at[0, 0], idxs[k], isem[k]).wait()

        def gather(c, k):
            pltpu.async_copy(h_hbm.at[idxs[k].at[0]], rows[k], gsem[k])

        def scatter(k):
            pltpu.async_copy(rows[k], acc_sh.at[idxs[k].at[1]], ssem[k],
                             add=True)

        for c in range(LI):
            fetch_idx(c, c)
        for c in range(LG):
            wait_idx(c)
            gather(c, c)
        plsc.subcore_barrier()

        @pl.loop(0, n_chunks, step=S)
        def _(g):
            for k in range(S):
                c = g + k
                ki = (k + LI) % S

                @pl.when(c + LI < n_chunks)
                def _():
                    @pl.when(c + LI - S >= 0)
                    def _():
                        pltpu.make_async_copy(rows[ki],
                                              acc_sh.at[idxs[ki].at[1]],
                                              ssem[ki]).wait()
                    fetch_idx(c + LI, ki)

                kg = (k + LG) % S

                @pl.when(c + LG < n_chunks)
                def _():
                    wait_idx(kg)
                    gather(c + LG, kg)

                pltpu.make_async_copy(h_hbm.at[idxs[k].at[0]], rows[k],
                                      gsem[k]).wait()
                scatter(k)

        for k in range(S):
            pltpu.make_async_copy(rows[k], acc_sh.at[idxs[k].at[1]],
                                  ssem[k]).wait()

        plsc.subcore_barrier()

        pltpu.sync_copy(
            acc_sh.at[pl.ds(row_base, rows_per_tile)],
            agg_out.at[pl.ds(cid * n_rows + row_base, rows_per_tile)])

    return body


def _sc_counts(n_rows, n_chunks):
    rows_per_tile = n_rows // NS
    mesh = plsc.VectorSubcoreMesh(core_axis_name="c", subcore_axis_name="s")

    @functools.partial(
        pl.kernel,
        out_type=jax.ShapeDtypeStruct((NC, n_rows, CW), jnp.float32),
        mesh=mesh,
        scratch_types=[
            pltpu.VMEM((n_chunks, CHUNK), jnp.int32),
            pltpu.VMEM((CHUNK, CW), jnp.float32),
            pltpu.VMEM_SHARED((n_rows, CW), jnp.float32),
        ],
        compiler_params=pltpu.CompilerParams(use_tc_tiling_on_sc=False))
    def body(dst_hbm, zc_hbm, ones_hbm, cnt_out, dst_v, ones_v, cnt_sh):
        cid = lax.axis_index("c")
        sid = lax.axis_index("s")
        wid = cid * NS + sid
        idx_base = wid * n_chunks
        row_base = sid * rows_per_tile

        pltpu.sync_copy(dst_hbm.at[pl.ds(idx_base, n_chunks)], dst_v)
        pltpu.sync_copy(ones_hbm, ones_v)
        pltpu.sync_copy(zc_hbm, cnt_sh.at[pl.ds(row_base, rows_per_tile)])
        plsc.subcore_barrier()

        @pl.loop(0, n_chunks)
        def _(j):
            pltpu.sync_copy(ones_v, cnt_sh.at[dst_v.at[j]], add=True)

        plsc.subcore_barrier()
        pltpu.sync_copy(cnt_sh.at[pl.ds(row_base, rows_per_tile)],
                        cnt_out.at[cid, pl.ds(row_base, rows_per_tile)])

    return body


def _tc_layer(n_rows, block):
    nb = n_rows // block

    def body(alo, ahi, xlo, xhi, cnt_ref, wl_ref, wr_ref, b_ref, o_ref):
        agg = jnp.concatenate([alo[...], ahi[...]], axis=1)
        x = jnp.concatenate([xlo[...], xhi[...]], axis=1)
        cnt = cnt_ref[0, :, 0:1] + cnt_ref[1, :, 0:1]
        mean = agg / jnp.maximum(cnt, 1.0)
        o_ref[...] = jnp.maximum(
            jnp.dot(mean, wl_ref[...], preferred_element_type=jnp.float32)
            + jnp.dot(x, wr_ref[...], preferred_element_type=jnp.float32)
            + b_ref[0:1, :], 0.0)

    half = pl.BlockSpec((block, DH), lambda i, j: (i, 0))
    other = pl.BlockSpec((block, DH), lambda i, j: (nb + i, 0))
    return pl.pallas_call(
        body,
        grid=(nb, 2),
        in_specs=[
            half, other,
            half, other,
            pl.BlockSpec((NC, block, CW), lambda i, j: (0, i, 0)),
            pl.BlockSpec((D, DH), lambda i, j: (j, 0)),
            pl.BlockSpec((D, DH), lambda i, j: (j, 0)),
            pl.BlockSpec((8, DH), lambda i, j: (j, 0)),
        ],
        out_specs=pl.BlockSpec((block, DH), lambda i, j: (j * nb + i, 0)),
        out_shape=jax.ShapeDtypeStruct((NC * n_rows, DH), jnp.float32),
    )


def kernel(x, edge_index, W_l0, b_l0, W_r0, W_l1, b_l1, W_r1):
    n, d = x.shape
    e = edge_index.shape[1]

    nc_cnt = -(-e // (NW * CHUNK))
    nc_cnt = -(-nc_cnt // 8) * 8
    e_pad = NW * nc_cnt * CHUNK
    nc_agg = e_pad // (NS * CA)
    assert e_pad == NS * nc_agg * CA
    rows_per_tile = -(-n // NW)
    rows_per_tile = -(-rows_per_tile // CHUNK) * CHUNK
    n_rows = rows_per_tile * NW

    src = edge_index[0].astype(jnp.int32)
    dst = edge_index[1].astype(jnp.int32)
    src = jnp.pad(src, (0, e_pad - e))
    dst = jnp.pad(dst, (0, e_pad - e), constant_values=n_rows - 1)
    srcc = src.reshape(NS * nc_agg, 1, CA)
    dstc = dst.reshape(NS * nc_agg, 1, CA)
    idxp = jnp.stack([
        jnp.concatenate([srcc, dstc], axis=1),
        jnp.concatenate([srcc + n_rows, dstc], axis=1),
    ])
    dst2d = dst.reshape(NW * nc_cnt, CHUNK)

    zeros = jnp.zeros((n_rows // NS, DH), jnp.float32)
    zeros_cnt = jnp.zeros((n_rows // NS, CW), jnp.float32)
    ones = jnp.ones((CHUNK, CW), jnp.float32)

    x_pad = jnp.pad(x, ((0, n_rows - n), (0, 0)))
    x_cat = jnp.concatenate([x_pad[:, :DH], x_pad[:, DH:]], axis=0)

    def colsplit(w):
        return jnp.concatenate([w[:, :DH], w[:, DH:]], axis=0)

    wl0, wr0 = colsplit(W_l0), colsplit(W_r0)
    wl1, wr1 = colsplit(W_l1), colsplit(W_r1)
    def biassplit(b):
        return jnp.pad(b.reshape(2, 1, DH),
                       ((0, 0), (0, 7), (0, 0))).reshape(16, DH)

    bc0 = biassplit(b_l0)
    bc1 = biassplit(b_l1)

    sc_agg = _sc_agg(n_rows, nc_agg)
    sc_counts = _sc_counts(n_rows, nc_cnt)
    tc = _tc_layer(n_rows, 512)

    cnt = sc_counts(dst2d, zeros_cnt, ones)
    agg0 = sc_agg(x_cat, idxp, zeros)
    h1 = tc(agg0, agg0, x_cat, x_cat, cnt, wl0, wr0, bc0)
    agg1 = sc_agg(h1, idxp, zeros)
    h2 = tc(agg1, agg1, h1, h1, cnt, wl1, wr1, bc1)
    return jnp.concatenate([h2[:n], h2[n_rows:n_rows + n]], axis=1)


# speedup vs baseline: 1.0970x; 1.0201x over previous
import functools

import jax
import jax.numpy as jnp
from jax import lax
from jax.experimental import pallas as pl
from jax.experimental.pallas import tpu as pltpu, tpu_sc as plsc

NC = 2
NS = 16
NW = NC * NS

D = 128
DH = D // 2
CHUNK = 128
CA = 128
CW = 8

PERM = [g + 16 * o + j for g in (0, 32) for j in range(16) for o in (0, 1)]


def _sc_agg(n_rows, n_chunks):
    rows_per_tile = n_rows // NS
    mesh = plsc.VectorSubcoreMesh(core_axis_name="c", subcore_axis_name="s")
    S = 6
    LG = 2
    LI = 4

    @functools.partial(
        pl.kernel,
        out_type=jax.ShapeDtypeStruct((NC * n_rows, DH), jnp.float32),
        mesh=mesh,
        scratch_types=[
            [pltpu.VMEM((2, CA), jnp.int32)] * S,
            [pltpu.VMEM((CA, DH), jnp.bfloat16)] * S,
            [pltpu.VMEM((CA, DH), jnp.float32)] * S,
            pltpu.VMEM_SHARED((n_rows, DH), jnp.float32),
            [pltpu.SemaphoreType.DMA] * S,
            [pltpu.SemaphoreType.DMA] * S,
            [pltpu.SemaphoreType.DMA] * S,
        ],
        compiler_params=pltpu.CompilerParams(use_tc_tiling_on_sc=False,
                                             needs_layout_passes=False))
    def body(h_hbm, idx_hbm, zeros_hbm, agg_out,
             idxs, brows, frows, acc_sh, isem, gsem, ssem):
        cid = lax.axis_index("c")
        sid = lax.axis_index("s")
        idx_base = sid * n_chunks
        row_base = sid * rows_per_tile

        pltpu.sync_copy(zeros_hbm.at[pl.ds(0, rows_per_tile)],
                        acc_sh.at[pl.ds(row_base, rows_per_tile)])

        def fetch_idx(c, k):
            pltpu.async_copy(idx_hbm.at[cid, idx_base + c], idxs[k], isem[k])

        def wait_idx(k):
            pltpu.make_async_copy(idx_hbm.at[0, 0], idxs[k], isem[k]).wait()

        def gather(c, k):
            pltpu.async_copy(h_hbm.at[idxs[k].at[0]], brows[k], gsem[k])

        def unpack(k):
            b, f = brows[k], frows[k]

            @pl.loop(0, CA)
            def _(r):
                for half in range(DH // 32):
                    i0 = plsc.bitcast(b[r, pl.ds(32 * half, 32)], jnp.int32)
                    lo = plsc.bitcast(lax.shift_left(i0, 16), jnp.float32)
                    hi = plsc.bitcast(
                        lax.bitwise_and(i0, jnp.int32(-65536)), jnp.float32)
                    f[r, pl.ds(32 * half, 16)] = lo
                    f[r, pl.ds(32 * half + 16, 16)] = hi

        def scatter(k):
            pltpu.async_copy(frows[k], acc_sh.at[idxs[k].at[1]], ssem[k],
                             add=True)

        for c in range(LI):
            fetch_idx(c, c)
        for c in range(LG):
            wait_idx(c)
            gather(c, c)
        plsc.subcore_barrier()

        @pl.loop(0, S * ((n_chunks + S - 1) // S), step=S)
        def _(g):
            for k in range(S):
                c = g + k
                ki = (k + LI) % S

                @pl.when(c + LI < n_chunks)
                def _():
                    @pl.when(c + LI - S >= 0)
                    def _():
                        pltpu.make_async_copy(frows[ki],
                                              acc_sh.at[idxs[ki].at[1]],
                                              ssem[ki]).wait()
                    fetch_idx(c + LI, ki)

                kg = (k + LG) % S

                @pl.when(c + LG < n_chunks)
                def _():
                    wait_idx(kg)
                    gather(c + LG, kg)

                @pl.when(c < n_chunks)
                def _():
                    pltpu.make_async_copy(h_hbm.at[idxs[k].at[0]], brows[k],
                                          gsem[k]).wait()
                    unpack(k)
                    scatter(k)

        for k in range(S):
            pltpu.make_async_copy(frows[k], acc_sh.at[idxs[k].at[1]],
                                  ssem[k]).wait()

        plsc.subcore_barrier()

        pltpu.sync_copy(
            acc_sh.at[pl.ds(row_base, rows_per_tile)],
            agg_out.at[pl.ds(cid * n_rows + row_base, rows_per_tile)])

    return body


def _sc_counts(n_rows, n_chunks):
    rows_per_tile = n_rows // NS
    mesh = plsc.VectorSubcoreMesh(core_axis_name="c", subcore_axis_name="s")

    @functools.partial(
        pl.kernel,
        out_type=jax.ShapeDtypeStruct((NC, n_rows, CW), jnp.float32),
        mesh=mesh,
        scratch_types=[
            pltpu.VMEM((n_chunks, CHUNK), jnp.int32),
            pltpu.VMEM((CHUNK, CW), jnp.float32),
            pltpu.VMEM_SHARED((n_rows, CW), jnp.float32),
        ],
        compiler_params=pltpu.CompilerParams(use_tc_tiling_on_sc=False))
    def body(dst_hbm, zc_hbm, ones_hbm, cnt_out, dst_v, ones_v, cnt_sh):
        cid = lax.axis_index("c")
        sid = lax.axis_index("s")
        wid = cid * NS + sid
        idx_base = wid * n_chunks
        row_base = sid * rows_per_tile

        pltpu.sync_copy(dst_hbm.at[pl.ds(idx_base, n_chunks)], dst_v)
        pltpu.sync_copy(ones_hbm, ones_v)
        pltpu.sync_copy(zc_hbm, cnt_sh.at[pl.ds(row_base, rows_per_tile)])
        plsc.subcore_barrier()

        @pl.loop(0, n_chunks)
        def _(j):
            pltpu.sync_copy(ones_v, cnt_sh.at[dst_v.at[j]], add=True)

        plsc.subcore_barrier()
        pltpu.sync_copy(cnt_sh.at[pl.ds(row_base, rows_per_tile)],
                        cnt_out.at[cid, pl.ds(row_base, rows_per_tile)])

    return body


def _tc_layer(n_rows, block):
    nb = n_rows // block

    def body(alo, ahi, xlo, xhi, cnt_ref, wl_ref, wr_ref, b_ref, o_ref):
        agg = jnp.concatenate([alo[...], ahi[...]], axis=1)
        x = jnp.concatenate([xlo[...], xhi[...]], axis=1)
        cnt = cnt_ref[0, :, 0:1] + cnt_ref[1, :, 0:1]
        mean = agg / jnp.maximum(cnt, 1.0)
        o_ref[...] = jnp.maximum(
            jnp.dot(mean, wl_ref[...], preferred_element_type=jnp.float32)
            + jnp.dot(x, wr_ref[...], preferred_element_type=jnp.float32)
            + b_ref[0:1, :], 0.0)

    half = pl.BlockSpec((block, DH), lambda i, j: (i, 0))
    other = pl.BlockSpec((block, DH), lambda i, j: (nb + i, 0))
    return pl.pallas_call(
        body,
        grid=(nb, 2),
        in_specs=[
            half, other,
            half, other,
            pl.BlockSpec((NC, block, CW), lambda i, j: (0, i, 0)),
            pl.BlockSpec((D, DH), lambda i, j: (j, 0)),
            pl.BlockSpec((D, DH), lambda i, j: (j, 0)),
            pl.BlockSpec((8, DH), lambda i, j: (j, 0)),
        ],
        out_specs=pl.BlockSpec((block, DH), lambda i, j: (j * nb + i, 0)),
        out_shape=jax.ShapeDtypeStruct((NC * n_rows, DH), jnp.float32),
    )


def kernel(x, edge_index, W_l0, b_l0, W_r0, W_l1, b_l1, W_r1):
    n, d = x.shape
    e = edge_index.shape[1]

    nc_cnt = -(-e // (NW * CHUNK))
    nc_cnt = -(-nc_cnt // 8) * 8
    e_pad = NW * nc_cnt * CHUNK
    nc_agg = e_pad // (NS * CA)
    assert e_pad == NS * nc_agg * CA
    rows_per_tile = -(-n // NW)
    rows_per_tile = -(-rows_per_tile // CHUNK) * CHUNK
    n_rows = rows_per_tile * NW

    src = edge_index[0].astype(jnp.int32)
    dst = edge_index[1].astype(jnp.int32)
    src = jnp.pad(src, (0, e_pad - e))
    dst = jnp.pad(dst, (0, e_pad - e), constant_values=n_rows - 1)
    srcc = src.reshape(NS * nc_agg, 1, CA)
    dstc = dst.reshape(NS * nc_agg, 1, CA)
    idxp = jnp.stack([
        jnp.concatenate([srcc, dstc], axis=1),
        jnp.concatenate([srcc + n_rows, dstc], axis=1),
    ])
    dst2d = dst.reshape(NW * nc_cnt, CHUNK)

    zeros = jnp.zeros((n_rows // NS, DH), jnp.float32)
    zeros_cnt = jnp.zeros((n_rows // NS, CW), jnp.float32)
    ones = jnp.ones((CHUNK, CW), jnp.float32)

    x_pad = jnp.pad(x, ((0, n_rows - n), (0, 0)))
    x_cat = jnp.concatenate([x_pad[:, :DH], x_pad[:, DH:]], axis=0)

    def colsplit(w):
        return jnp.concatenate([w[:, :DH], w[:, DH:]], axis=0)

    wl0, wr0 = colsplit(W_l0), colsplit(W_r0)
    wl1, wr1 = colsplit(W_l1), colsplit(W_r1)
    def biassplit(b):
        return jnp.pad(b.reshape(2, 1, DH),
                       ((0, 0), (0, 7), (0, 0))).reshape(16, DH)

    bc0 = biassplit(b_l0)
    bc1 = biassplit(b_l1)

    sc_agg = _sc_agg(n_rows, nc_agg)
    sc_counts = _sc_counts(n_rows, nc_cnt)
    tc = _tc_layer(n_rows, 512)

    perm = jnp.array(PERM, jnp.int32)
    cnt = sc_counts(dst2d, zeros_cnt, ones)
    agg0 = sc_agg(x_cat[:, perm].astype(jnp.bfloat16), idxp, zeros)
    h1 = tc(agg0, agg0, x_cat, x_cat, cnt, wl0, wr0, bc0)
    agg1 = sc_agg(h1[:, perm].astype(jnp.bfloat16), idxp, zeros)
    h2 = tc(agg1, agg1, h1, h1, cnt, wl1, wr1, bc1)
    return jnp.concatenate([h2[:n], h2[n_rows:n_rows + n]], axis=1)


# speedup vs baseline: 1.1008x; 1.0034x over previous
import functools

import jax
import jax.numpy as jnp
from jax import lax
from jax.experimental import pallas as pl
from jax.experimental.pallas import tpu as pltpu, tpu_sc as plsc

NC = 2
NS = 16
NW = NC * NS

D = 128
DH = D // 2
CHUNK = 128
CA = 128
CW = 8

PERM = [g + 16 * o + j for g in (0, 32) for j in range(16) for o in (0, 1)]


def _sc_agg(n_rows, n_chunks):
    rows_per_tile = n_rows // NS
    mesh = plsc.VectorSubcoreMesh(core_axis_name="c", subcore_axis_name="s")
    S = 6
    LG = 3
    LI = 4

    @functools.partial(
        pl.kernel,
        out_type=jax.ShapeDtypeStruct((NC * n_rows, DH), jnp.float32),
        mesh=mesh,
        scratch_types=[
            [pltpu.VMEM((2, CA), jnp.int32)] * S,
            [pltpu.VMEM((CA, DH), jnp.bfloat16)] * S,
            [pltpu.VMEM((CA, DH), jnp.float32)] * S,
            pltpu.VMEM_SHARED((n_rows, DH), jnp.float32),
            [pltpu.SemaphoreType.DMA] * S,
            [pltpu.SemaphoreType.DMA] * S,
            [pltpu.SemaphoreType.DMA] * S,
        ],
        compiler_params=pltpu.CompilerParams(use_tc_tiling_on_sc=False,
                                             needs_layout_passes=False))
    def body(h_hbm, idx_hbm, zeros_hbm, agg_out,
             idxs, brows, frows, acc_sh, isem, gsem, ssem):
        cid = lax.axis_index("c")
        sid = lax.axis_index("s")
        idx_base = sid * n_chunks
        row_base = sid * rows_per_tile

        pltpu.sync_copy(zeros_hbm.at[pl.ds(0, rows_per_tile)],
                        acc_sh.at[pl.ds(row_base, rows_per_tile)])

        def fetch_idx(c, k):
            pltpu.async_copy(idx_hbm.at[cid, idx_base + c], idxs[k], isem[k])

        def wait_idx(k):
            pltpu.make_async_copy(idx_hbm.at[0, 0], idxs[k], isem[k]).wait()

        def gather(c, k):
            pltpu.async_copy(h_hbm.at[idxs[k].at[0]], brows[k], gsem[k])

        def unpack(k):
            b, f = brows[k], frows[k]

            @pl.loop(0, CA)
            def _(r):
                for half in range(DH // 32):
                    i0 = plsc.bitcast(b[r, pl.ds(32 * half, 32)], jnp.int32)
                    lo = plsc.bitcast(lax.shift_left(i0, 16), jnp.float32)
                    hi = plsc.bitcast(
                        lax.bitwise_and(i0, jnp.int32(-65536)), jnp.float32)
                    f[r, pl.ds(32 * half, 16)] = lo
                    f[r, pl.ds(32 * half + 16, 16)] = hi

        def scatter(k):
            pltpu.async_copy(frows[k], acc_sh.at[idxs[k].at[1]], ssem[k],
                             add=True)

        for c in range(LI):
            fetch_idx(c, c)
        for c in range(LG):
            wait_idx(c)
            gather(c, c)
        plsc.subcore_barrier()

        @pl.loop(0, S * ((n_chunks + S - 1) // S), step=S)
        def _(g):
            for k in range(S):
                c = g + k
                ki = (k + LI) % S

                @pl.when(c + LI < n_chunks)
                def _():
                    @pl.when(c + LI - S >= 0)
                    def _():
                        pltpu.make_async_copy(frows[ki],
                                              acc_sh.at[idxs[ki].at[1]],
                                              ssem[ki]).wait()
                    fetch_idx(c + LI, ki)

                kg = (k + LG) % S

                @pl.when(c + LG < n_chunks)
                def _():
                    wait_idx(kg)
                    gather(c + LG, kg)

                @pl.when(c < n_chunks)
                def _():
                    pltpu.make_async_copy(h_hbm.at[idxs[k].at[0]], brows[k],
                                          gsem[k]).wait()
                    unpack(k)
                    scatter(k)

        for k in range(S):
            pltpu.make_async_copy(frows[k], acc_sh.at[idxs[k].at[1]],
                                  ssem[k]).wait()

        plsc.subcore_barrier()

        pltpu.sync_copy(
            acc_sh.at[pl.ds(row_base, rows_per_tile)],
            agg_out.at[pl.ds(cid * n_rows + row_base, rows_per_tile)])

    return body


def _sc_counts(n_rows, n_chunks):
    rows_per_tile = n_rows // NS
    mesh = plsc.VectorSubcoreMesh(core_axis_name="c", subcore_axis_name="s")

    @functools.partial(
        pl.kernel,
        out_type=jax.ShapeDtypeStruct((NC, n_rows, CW), jnp.float32),
        mesh=mesh,
        scratch_types=[
            pltpu.VMEM((n_chunks, CHUNK), jnp.int32),
            pltpu.VMEM((CHUNK, CW), jnp.float32),
            pltpu.VMEM_SHARED((n_rows, CW), jnp.float32),
        ],
        compiler_params=pltpu.CompilerParams(use_tc_tiling_on_sc=False))
    def body(dst_hbm, zc_hbm, ones_hbm, cnt_out, dst_v, ones_v, cnt_sh):
        cid = lax.axis_index("c")
        sid = lax.axis_index("s")
        wid = cid * NS + sid
        idx_base = wid * n_chunks
        row_base = sid * rows_per_tile

        pltpu.sync_copy(dst_hbm.at[pl.ds(idx_base, n_chunks)], dst_v)
        pltpu.sync_copy(ones_hbm, ones_v)
        pltpu.sync_copy(zc_hbm, cnt_sh.at[pl.ds(row_base, rows_per_tile)])
        plsc.subcore_barrier()

        @pl.loop(0, n_chunks)
        def _(j):
            pltpu.sync_copy(ones_v, cnt_sh.at[dst_v.at[j]], add=True)

        plsc.subcore_barrier()
        pltpu.sync_copy(cnt_sh.at[pl.ds(row_base, rows_per_tile)],
                        cnt_out.at[cid, pl.ds(row_base, rows_per_tile)])

    return body


def _tc_layer(n_rows, block):
    nb = n_rows // block

    def body(alo, ahi, xlo, xhi, cnt_ref, wl_ref, wr_ref, b_ref, o_ref):
        agg = jnp.concatenate([alo[...], ahi[...]], axis=1)
        x = jnp.concatenate([xlo[...], xhi[...]], axis=1)
        cnt = cnt_ref[0, :, 0:1] + cnt_ref[1, :, 0:1]
        mean = agg / jnp.maximum(cnt, 1.0)
        o_ref[...] = jnp.maximum(
            jnp.dot(mean, wl_ref[...], preferred_element_type=jnp.float32)
            + jnp.dot(x, wr_ref[...], preferred_element_type=jnp.float32)
            + b_ref[0:1, :], 0.0)

    half = pl.BlockSpec((block, DH), lambda i, j: (i, 0))
    other = pl.BlockSpec((block, DH), lambda i, j: (nb + i, 0))
    return pl.pallas_call(
        body,
        grid=(nb, 2),
        in_specs=[
            half, other,
            half, other,
            pl.BlockSpec((NC, block, CW), lambda i, j: (0, i, 0)),
            pl.BlockSpec((D, DH), lambda i, j: (j, 0)),
            pl.BlockSpec((D, DH), lambda i, j: (j, 0)),
            pl.BlockSpec((8, DH), lambda i, j: (j, 0)),
        ],
        out_specs=pl.BlockSpec((block, DH), lambda i, j: (j * nb + i, 0)),
        out_shape=jax.ShapeDtypeStruct((NC * n_rows, DH), jnp.float32),
    )


def kernel(x, edge_index, W_l0, b_l0, W_r0, W_l1, b_l1, W_r1):
    n, d = x.shape
    e = edge_index.shape[1]

    nc_cnt = -(-e // (NW * CHUNK))
    nc_cnt = -(-nc_cnt // 8) * 8
    e_pad = NW * nc_cnt * CHUNK
    nc_agg = e_pad // (NS * CA)
    assert e_pad == NS * nc_agg * CA
    rows_per_tile = -(-n // NW)
    rows_per_tile = -(-rows_per_tile // CHUNK) * CHUNK
    n_rows = rows_per_tile * NW

    src = edge_index[0].astype(jnp.int32)
    dst = edge_index[1].astype(jnp.int32)
    src = jnp.pad(src, (0, e_pad - e))
    dst = jnp.pad(dst, (0, e_pad - e), constant_values=n_rows - 1)
    srcc = src.reshape(NS * nc_agg, 1, CA)
    dstc = dst.reshape(NS * nc_agg, 1, CA)
    idxp = jnp.stack([
        jnp.concatenate([srcc, dstc], axis=1),
        jnp.concatenate([srcc + n_rows, dstc], axis=1),
    ])
    dst2d = dst.reshape(NW * nc_cnt, CHUNK)

    zeros = jnp.zeros((n_rows // NS, DH), jnp.float32)
    zeros_cnt = jnp.zeros((n_rows // NS, CW), jnp.float32)
    ones = jnp.ones((CHUNK, CW), jnp.float32)

    x_pad = jnp.pad(x, ((0, n_rows - n), (0, 0)))
    x_cat = jnp.concatenate([x_pad[:, :DH], x_pad[:, DH:]], axis=0)

    def colsplit(w):
        return jnp.concatenate([w[:, :DH], w[:, DH:]], axis=0)

    wl0, wr0 = colsplit(W_l0), colsplit(W_r0)
    wl1, wr1 = colsplit(W_l1), colsplit(W_r1)
    def biassplit(b):
        return jnp.pad(b.reshape(2, 1, DH),
                       ((0, 0), (0, 7), (0, 0))).reshape(16, DH)

    bc0 = biassplit(b_l0)
    bc1 = biassplit(b_l1)

    sc_agg = _sc_agg(n_rows, nc_agg)
    sc_counts = _sc_counts(n_rows, nc_cnt)
    tc = _tc_layer(n_rows, 512)

    perm = jnp.array(PERM, jnp.int32)
    cnt = sc_counts(dst2d, zeros_cnt, ones)
    agg0 = sc_agg(x_cat[:, perm].astype(jnp.bfloat16), idxp, zeros)
    h1 = tc(agg0, agg0, x_cat, x_cat, cnt, wl0, wr0, bc0)
    agg1 = sc_agg(h1[:, perm].astype(jnp.bfloat16), idxp, zeros)
    h2 = tc(agg1, agg1, h1, h1, cnt, wl1, wr1, bc1)
    return jnp.concatenate([h2[:n], h2[n_rows:n_rows + n]], axis=1)
